# baseline (speedup 1.0000x reference)
import jax
import jax.numpy as jnp
from jax.experimental import pallas as pl
from jax.experimental.pallas import tpu as pltpu

_LANES = 128
_ITEM_CHUNKS = 8
_USER_CHUNKS = 1


def _copy_body(u_in, i_in, u_out, i_out, sems):
    copies = []
    rows_u = u_in.shape[0] // _USER_CHUNKS
    for c in range(_USER_CHUNKS):
        copies.append(pltpu.make_async_copy(
            u_in.at[pl.ds(c * rows_u, rows_u)],
            u_out.at[pl.ds(c * rows_u, rows_u)],
            sems.at[c],
        ))
    rows_i = i_in.shape[0] // _ITEM_CHUNKS
    for c in range(_ITEM_CHUNKS):
        copies.append(pltpu.make_async_copy(
            i_in.at[pl.ds(c * rows_i, rows_i)],
            i_out.at[pl.ds(c * rows_i, rows_i)],
            sems.at[_USER_CHUNKS + c],
        ))
    for cp in copies:
        cp.start()
    for cp in copies:
        cp.wait()


def kernel(user_weight, item_weight, edge_index):
    nu, d = user_weight.shape
    ni, _ = item_weight.shape
    u2 = user_weight.reshape(nu * d // _LANES, _LANES)
    i2 = item_weight.reshape(ni * d // _LANES, _LANES)
    out_shape = (
        jax.ShapeDtypeStruct(u2.shape, u2.dtype),
        jax.ShapeDtypeStruct(i2.shape, i2.dtype),
    )
    uo, io = pl.pallas_call(
        _copy_body,
        in_specs=[
            pl.BlockSpec(memory_space=pl.ANY),
            pl.BlockSpec(memory_space=pl.ANY),
        ],
        out_specs=(
            pl.BlockSpec(memory_space=pl.ANY),
            pl.BlockSpec(memory_space=pl.ANY),
        ),
        out_shape=out_shape,
        scratch_shapes=[pltpu.SemaphoreType.DMA((_USER_CHUNKS + _ITEM_CHUNKS,))],
    )(u2, i2)
    return uo.reshape(nu, d), io.reshape(ni, d)


# speedup vs baseline: 4.6064x; 4.6064x over previous
import jax
import jax.numpy as jnp
from jax.experimental import pallas as pl
from jax.experimental.pallas import tpu as pltpu

_LANES = 128
_GRID = 25


def _copy_body(u_in, i_in, u_out, i_out):
    u_out[...] = u_in[...]
    i_out[...] = i_in[...]


def kernel(user_weight, item_weight, edge_index):
    nu, d = user_weight.shape
    ni, _ = item_weight.shape
    u2 = user_weight.reshape(nu * d // _LANES, _LANES)
    i2 = item_weight.reshape(ni * d // _LANES, _LANES)
    bu = u2.shape[0] // _GRID
    bi = i2.shape[0] // _GRID
    out_shape = (
        jax.ShapeDtypeStruct(u2.shape, u2.dtype),
        jax.ShapeDtypeStruct(i2.shape, i2.dtype),
    )
    uo, io = pl.pallas_call(
        _copy_body,
        grid=(_GRID,),
        in_specs=[
            pl.BlockSpec((bu, _LANES), lambda g: (g, 0)),
            pl.BlockSpec((bi, _LANES), lambda g: (g, 0)),
        ],
        out_specs=(
            pl.BlockSpec((bu, _LANES), lambda g: (g, 0)),
            pl.BlockSpec((bi, _LANES), lambda g: (g, 0)),
        ),
        out_shape=out_shape,
        compiler_params=pltpu.CompilerParams(
            dimension_semantics=("arbitrary",),
        ),
    )(u2, i2)
    return uo.reshape(nu, d), io.reshape(ni, d)


# speedup vs baseline: 4.8363x; 1.0499x over previous
import functools

import jax
import jax.numpy as jnp
from jax import lax
from jax.experimental import pallas as pl
from jax.experimental.pallas import tpu as pltpu
from jax.experimental.pallas import tpu_sc as plsc

_NC, _NS = 2, 16
_NW = _NC * _NS

_U_ROWS, _I_ROWS, _DIM = 100000, 1000000, 32
_U_CHUNK = 200
_I_CHUNK = 320
_NBUF = 3


def _phase(src, dst, n_rows, chunk, wid, bufs, sin, sout):
    n_chunks = n_rows // chunk
    j_max = -(-n_chunks // _NW)
    n_groups = -(-j_max // _NBUF)

    def pred(j):
        return (j * _NW + wid) < n_chunks

    def base(j):
        return pl.multiple_of((j * _NW + wid) * chunk, 8)

    def in_copy(j, b):
        return pltpu.make_async_copy(
            src.at[pl.ds(base(j), chunk)], bufs[b].at[pl.ds(0, chunk)], sin[b])

    def out_copy(j, b):
        return pltpu.make_async_copy(
            bufs[b].at[pl.ds(0, chunk)], dst.at[pl.ds(base(j), chunk)], sout[b])

    def group(g, carry):
        for b in range(_NBUF):
            j = g * _NBUF + b

            @pl.when(pred(j))
            def _():
                pltpu.sync_copy(src.at[pl.ds(base(j), chunk)],
                                bufs[b].at[pl.ds(0, chunk)])
                pltpu.sync_copy(bufs[b].at[pl.ds(0, chunk)],
                                dst.at[pl.ds(base(j), chunk)])
        return carry

    lax.fori_loop(0, n_groups, group, 0)


def _sc_copy_body(u_in, i_in, u_out, i_out,
                  buf0, buf1, buf2, si0, si1, si2, so0, so1, so2):
    wid = lax.axis_index("s") * _NC + lax.axis_index("c")
    bufs = (buf0, buf1, buf2)
    sin = (si0, si1, si2)
    sout = (so0, so1, so2)
    _phase(i_in, i_out, _I_ROWS, _I_CHUNK, wid, bufs, sin, sout)
    _phase(u_in, u_out, _U_ROWS, _U_CHUNK, wid, bufs, sin, sout)


@functools.partial(
    pl.kernel,
    out_type=(
        jax.ShapeDtypeStruct((_U_ROWS, _DIM), jnp.float32),
        jax.ShapeDtypeStruct((_I_ROWS, _DIM), jnp.float32),
    ),
    mesh=plsc.VectorSubcoreMesh(core_axis_name="c", subcore_axis_name="s"),
    scratch_types=[
        pltpu.VMEM((_I_CHUNK, _DIM), jnp.float32),
        pltpu.VMEM((_I_CHUNK, _DIM), jnp.float32),
        pltpu.VMEM((_I_CHUNK, _DIM), jnp.float32),
        pltpu.SemaphoreType.DMA,
        pltpu.SemaphoreType.DMA,
        pltpu.SemaphoreType.DMA,
        pltpu.SemaphoreType.DMA,
        pltpu.SemaphoreType.DMA,
        pltpu.SemaphoreType.DMA,
    ],
)
def _sc_copy(u_in, i_in, u_out, i_out, *scratch):
    _sc_copy_body(u_in, i_in, u_out, i_out, *scratch)


def kernel(user_weight, item_weight, edge_index):
    return _sc_copy(user_weight, item_weight)


# speedup vs baseline: 5.1172x; 1.0581x over previous
import functools

import jax
import jax.numpy as jnp
from jax import lax
from jax.experimental import pallas as pl
from jax.experimental.pallas import tpu as pltpu
from jax.experimental.pallas import tpu_sc as plsc

_NC, _NS = 2, 16
_NW = _NC * _NS

_U_ROWS, _I_ROWS, _DIM = 100000, 1000000, 32
_U_CHUNK = 200
_I_CHUNK = 320
_NBUF = 3


def _phase(src, dst, n_rows, chunk, wid, bufs, sin, sout):
    n_chunks = n_rows // chunk
    j_max = -(-n_chunks // _NW)
    n_groups = -(-j_max // _NBUF)

    def pred(j):
        return (j * _NW + wid) < n_chunks

    def base(j):
        return pl.multiple_of((j * _NW + wid) * chunk, 8)

    def in_copy(j, b):
        return pltpu.make_async_copy(
            src.at[pl.ds(base(j), chunk)], bufs[b].at[pl.ds(0, chunk)], sin[b])

    def out_copy(j, b):
        return pltpu.make_async_copy(
            bufs[b].at[pl.ds(0, chunk)], dst.at[pl.ds(base(j), chunk)], sout[b])

    def group(g, carry):
        for b in range(_NBUF):
            j = g * _NBUF + b
            pl.when(pred(j))(in_copy(j, b).start)
        for b in range(_NBUF):
            j = g * _NBUF + b

            @pl.when(pred(j))
            def _():
                in_copy(j, b).wait()
                out_copy(j, b).start()
        for b in range(_NBUF):
            j = g * _NBUF + b
            pl.when(pred(j))(out_copy(j, b).wait)
        return carry

    lax.fori_loop(0, n_groups, group, 0)


def _sc_copy_body(u_in, i_in, u_out, i_out,
                  buf0, buf1, buf2, si0, si1, si2, so0, so1, so2):
    wid = lax.axis_index("s") * _NC + lax.axis_index("c")
    bufs = (buf0, buf1, buf2)
    sin = (si0, si1, si2)
    sout = (so0, so1, so2)
    _phase(i_in, i_out, _I_ROWS, _I_CHUNK, wid, bufs, sin, sout)
    _phase(u_in, u_out, _U_ROWS, _U_CHUNK, wid, bufs, sin, sout)


@functools.partial(
    pl.kernel,
    out_type=(
        jax.ShapeDtypeStruct((_U_ROWS, _DIM), jnp.float32),
        jax.ShapeDtypeStruct((_I_ROWS, _DIM), jnp.float32),
    ),
    mesh=plsc.VectorSubcoreMesh(core_axis_name="c", subcore_axis_name="s"),
    scratch_types=[
        pltpu.VMEM((_I_CHUNK, _DIM), jnp.float32),
        pltpu.VMEM((_I_CHUNK, _DIM), jnp.float32),
        pltpu.VMEM((_I_CHUNK, _DIM), jnp.float32),
        pltpu.SemaphoreType.DMA,
        pltpu.SemaphoreType.DMA,
        pltpu.SemaphoreType.DMA,
        pltpu.SemaphoreType.DMA,
        pltpu.SemaphoreType.DMA,
        pltpu.SemaphoreType.DMA,
    ],
)
def _sc_copy(u_in, i_in, u_out, i_out, *scratch):
    _sc_copy_body(u_in, i_in, u_out, i_out, *scratch)


def kernel(user_weight, item_weight, edge_index):
    return _sc_copy(user_weight, item_weight)
